# baseline (speedup 1.0000x reference)
import jax
import jax.numpy as jnp
from jax import lax
from jax.experimental import pallas as pl
from jax.experimental.pallas import tpu as pltpu
from jax.experimental.pallas import tpu_sc as plsc

D = 768
L = 16
NB = 16
NS_ = (1024, 256, 64)
OFF = (0, 1024, 1280)
NN = (32, 8, 2)
BIAS_OFF = (0, 32, 40)
RTOT = 1344
UNROLL = 8


def _add_rows(dst, src, src_r0, rows):
    @pl.loop(0, rows)
    def _(r):
        @plsc.parallel_loop(0, D, step=L, unroll=UNROLL)
        def _(v):
            plsc.addupdate(dst.at[pl.ds(r, 1), pl.ds(v, L)],
                           src.at[pl.ds(src_r0 + r, 1), pl.ds(v, L)][...])


def _sc_body(f0, f1, f2, sc_emb, patch, out, bias, srow, work,
             sin0, sin1, sout0, sout1):
    wid = lax.axis_index("subcore") * 2 + lax.axis_index("core")
    pltpu.sync_copy(sc_emb, srow)
    for i in range(3):
        n0 = wid * NN[i]
        pltpu.sync_copy(patch.at[pl.ds(2048 * i + n0, NN[i])],
                        bias.at[pl.ds(BIAS_OFF[i], NN[i])])
    for i in range(3):
        @pl.loop(0, NN[i])
        def _(r, i=i):
            @plsc.parallel_loop(0, D, step=L, unroll=UNROLL)
            def _(v, r=r, i=i):
                plsc.addupdate(bias.at[pl.ds(BIAS_OFF[i] + r, 1), pl.ds(v, L)],
                               srow.at[pl.ds(i, 1), pl.ds(v, L)][...])

    for i, fref in enumerate((f0, f1, f2)):
        nn = NN[i]
        n0 = wid * nn

        def fsl(b, i=i, nn=nn, n0=n0, fref=fref):
            return fref.at[pl.ds(b * NS_[i] + n0, nn)]

        def osl(b, i=i, nn=nn, n0=n0):
            return out.at[pl.ds(b * RTOT + OFF[i] + n0, nn)]

        buf0 = work.at[0].at[pl.ds(0, nn)]
        buf1 = work.at[1].at[pl.ds(0, nn)]

        @pl.loop(0, NB, step=2)
        def _(b, i=i, nn=nn, fsl=fsl, osl=osl, buf0=buf0, buf1=buf1):
            @pl.when(b > 0)
            def _():
                pltpu.make_async_copy(buf0, osl(b - 2), sout0).wait()
                pltpu.make_async_copy(buf1, osl(b - 1), sout1).wait()
            pltpu.make_async_copy(fsl(b), buf0, sin0).start()
            pltpu.make_async_copy(fsl(b + 1), buf1, sin1).start()
            pltpu.make_async_copy(fsl(b), buf0, sin0).wait()
            _add_rows(buf0, bias, BIAS_OFF[i], nn)
            pltpu.make_async_copy(buf0, osl(b), sout0).start()
            pltpu.make_async_copy(fsl(b + 1), buf1, sin1).wait()
            _add_rows(buf1, bias, BIAS_OFF[i], nn)
            pltpu.make_async_copy(buf1, osl(b + 1), sout1).start()

        pltpu.make_async_copy(buf0, osl(NB - 2), sout0).wait()
        pltpu.make_async_copy(buf1, osl(NB - 1), sout1).wait()


def kernel(features_per_scale_0, features_per_scale_1, features_per_scale_2,
           scale_embeddings, patch_embeddings):
    f0 = features_per_scale_0.reshape(NB * NS_[0], D)
    f1 = features_per_scale_1.reshape(NB * NS_[1], D)
    f2 = features_per_scale_2.reshape(NB * NS_[2], D)
    patch = patch_embeddings.reshape(3 * 2048, D)

    mesh = plsc.VectorSubcoreMesh(core_axis_name="core",
                                  subcore_axis_name="subcore")
    run = pl.kernel(
        _sc_body,
        out_type=jax.ShapeDtypeStruct((NB * RTOT, D), jnp.float32),
        mesh=mesh,
        scratch_types=[
            pltpu.VMEM((42, D), jnp.float32),
            pltpu.VMEM((3, D), jnp.float32),
            pltpu.VMEM((2, 32, D), jnp.float32),
            pltpu.SemaphoreType.DMA,
            pltpu.SemaphoreType.DMA,
            pltpu.SemaphoreType.DMA,
            pltpu.SemaphoreType.DMA,
        ],
    )
    out = run(f0, f1, f2, scale_embeddings, patch)
    return out.reshape(NB, RTOT, D)
